# Initial kernel scaffold; baseline (speedup 1.0000x reference)
import jax
import jax.numpy as jnp
from jax.experimental import pallas as pl


def kernel(x, weight, gamma, beta):
    raise NotImplementedError("write your pallas kernel here")


# speedup vs baseline: 1.0602x; 1.0602x over previous
import functools

import jax
import jax.numpy as jnp
from jax import lax
from jax.experimental import pallas as pl
from jax.experimental.pallas import tpu as pltpu
from jax.experimental.pallas import tpu_sc as plsc

EMB_N = 8192
EMB_D = 32
N_TOK = 16384
BN_EPS = 1e-5

TB = 256
NB = N_TOK // TB

NC = 2
NS = 16
NW = NC * NS
BPW = N_TOK // NW
IDX_CH = 128
CH_PER_W = BPW // IDX_CH


def _argmin_body(xb_ref, w_ref, idx_ref):
    xb = xb_ref[...]
    w = w_ref[...]
    x2 = jnp.sum(xb * xb, axis=1, keepdims=True)
    w2 = jnp.sum(w * w, axis=1)[None, :]
    s = lax.dot_general(xb, w, (((1,), (1,)), ((), ())))
    dist = x2 + w2 - 2.0 * s
    m = jnp.min(dist, axis=1, keepdims=True)
    col = lax.broadcasted_iota(jnp.int32, (TB, EMB_N), 1)
    ids = jnp.where(dist == m, col, jnp.int32(2**31 - 1))
    idx_ref[...] = jnp.min(ids, axis=1, keepdims=True)


def _nearest_idx(xb, weight):
    return pl.pallas_call(
        _argmin_body,
        grid=(NB,),
        in_specs=[
            pl.BlockSpec((TB, EMB_D), lambda i: (i, 0)),
            pl.BlockSpec((EMB_N, EMB_D), lambda i: (0, 0)),
        ],
        out_specs=pl.BlockSpec((TB, 1), lambda i: (i, 0)),
        out_shape=jax.ShapeDtypeStruct((N_TOK, 1), jnp.int32),
        compiler_params=pltpu.CompilerParams(
            dimension_semantics=("arbitrary",),
        ),
    )(xb, weight)


@functools.cache
def _sc_gather_fn():
    mesh = plsc.VectorSubcoreMesh(core_axis_name="c", subcore_axis_name="s")

    @functools.partial(
        pl.kernel,
        mesh=mesh,
        out_type=jax.ShapeDtypeStruct((N_TOK, EMB_D), jnp.float32),
        scratch_types=[
            pltpu.VMEM((CH_PER_W, IDX_CH), jnp.int32),
            pltpu.VMEM((BPW, EMB_D), jnp.float32),
            pltpu.SemaphoreType.DMA,
        ],
        compiler_params=pltpu.CompilerParams(use_tc_tiling_on_sc=False),
    )
    def _sc_gather(idx_hbm, table_hbm, out_hbm, idx_v, rows_v, sem):
        wid = lax.axis_index("s") * NC + lax.axis_index("c")
        base = wid * BPW
        pltpu.sync_copy(idx_hbm.at[pl.ds(wid * CH_PER_W, CH_PER_W)], idx_v)
        copies = []
        for j in range(CH_PER_W):
            copies.append(
                pltpu.async_copy(
                    table_hbm.at[idx_v.at[j]],
                    rows_v.at[pl.ds(j * IDX_CH, IDX_CH)],
                    sem,
                )
            )
        for cp in copies:
            cp.wait()
        pltpu.sync_copy(rows_v, out_hbm.at[pl.ds(base, BPW)])

    return _sc_gather


def kernel(x, weight, gamma, beta):
    mean = jnp.mean(x, axis=0)
    var = jnp.var(x, axis=0)
    xb = (x - mean) / jnp.sqrt(var + BN_EPS) * gamma + beta
    idx = _nearest_idx(xb, weight)
    idx2d = idx.reshape(N_TOK // IDX_CH, IDX_CH)
    return _sc_gather_fn()(idx2d, weight)
